# baseline (speedup 1.0000x reference)
import functools

import jax
import jax.numpy as jnp
import numpy as np
from jax import lax
from jax.scipy.special import erf
from jax.experimental import pallas as pl
from jax.experimental.pallas import tpu as pltpu
from jax.experimental.pallas import tpu_sc as plsc

N = 2048
DIM_U = 16
TEMPERATURE = 0.3
LOG2 = 0.69314718056
P = N * (N - 1) // 2

NC, NS, L = 2, 16, 16
NW = NC * NS
ROWS_W = N // NW
WIN = N + 8

_MESH = plsc.VectorSubcoreMesh(core_axis_name="c", subcore_axis_name="s")
_SC_PARAMS = pltpu.CompilerParams(needs_layout_passes=False)


def _win_start(i):
    v = i * (N - 1) - ((i * (i - 1)) >> 1) - i - 1
    w8 = jnp.maximum(jnp.minimum(v & -8, P - WIN), 0)
    w8 = pl.multiple_of(w8, 8)
    return w8, v - w8


@functools.partial(
    pl.kernel,
    mesh=_MESH,
    compiler_params=_SC_PARAMS,
    out_type=jax.ShapeDtypeStruct((N, N), jnp.float32),
    scratch_types=[
        pltpu.VMEM((4 * WIN,), jnp.float32),
        pltpu.VMEM((4 * N,), jnp.float32),
        pltpu.SemaphoreType.DMA,
        pltpu.SemaphoreType.DMA,
    ],
)
def _noise_shear(noise_hbm, nm_hbm, win_v, row_v, sem_in, sem_out):
    wid = lax.axis_index("s") * NC + lax.axis_index("c")
    base = pl.multiple_of(wid * ROWS_W, ROWS_W)
    RING = 4

    def wslice(b):
        return win_v.at[pl.ds(pl.multiple_of(b * WIN, 8), WIN)]

    def rslice(b):
        return row_v.at[pl.ds(pl.multiple_of(b * N, 8), N)]

    for k in range(RING - 1):
        w8k, _ = _win_start(base + k)
        pltpu.async_copy(noise_hbm.at[pl.ds(w8k, WIN)], wslice(k), sem_in)

    def row_body(r, carry):
        i = base + r
        slot = lax.rem(r, RING)
        pslot = lax.rem(r + RING - 1, RING)
        w8n, _ = _win_start(i + RING - 1)
        pltpu.async_copy(noise_hbm.at[pl.ds(w8n, WIN)], wslice(pslot), sem_in)
        pltpu.make_async_copy(
            noise_hbm.at[pl.ds(w8n, WIN)], wslice(slot), sem_in).wait()

        @pl.when(r >= RING)
        def _():
            pltpu.make_async_copy(rslice(slot), nm_hbm.at[i - RING],
                                  sem_out).wait()

        _, dlt = _win_start(i)
        win = wslice(slot)
        row = rslice(slot)

        off0 = jnp.maximum(lax.iota(jnp.int32, L) + dlt, 0)
        row[pl.ds(0, L)] = plsc.load_gather(win, [off0])

        @plsc.parallel_loop(L, N, L, unroll=8)
        def _shift(o):
            row[pl.ds(o, L)] = win[pl.ds(o + dlt, L)]

        pltpu.async_copy(row, nm_hbm.at[i], sem_out)
        return carry

    lax.fori_loop(0, ROWS_W, row_body, 0)
    for k in range(RING - 1):
        pltpu.make_async_copy(
            noise_hbm.at[pl.ds(0, WIN)], wslice(k), sem_in).wait()
    for k in range(RING):
        pltpu.make_async_copy(rslice(k), nm_hbm.at[base + k], sem_out).wait()


BR, BC = 256, 512


def _dense_body(s_ref, y_ref, yc_ref, nm_ref, o_ref):
    bi = pl.program_id(0)
    bj = pl.program_id(1)
    below = (bj + 1) * BC <= bi * BR

    @pl.when(below)
    def _():
        o_ref[...] = jnp.zeros((BR, BC), jnp.float32)

    @pl.when(jnp.logical_not(below))
    def _():
        inv2s = s_ref[0, 0]
        y = y_ref[...]
        yc = yc_ref[...]
        ny = jnp.sum(y * y, axis=1, keepdims=True)
        nyc = jnp.sum(yc * yc, axis=1)[None, :]
        dot = lax.dot_general(y, yc, (((1,), (1,)), ((), ())),
                              preferred_element_type=jnp.float32)
        d2 = jnp.maximum(ny + nyc - 2.0 * dot, 0.0)
        logp = d2 * inv2s
        logits = logp - jnp.log(jnp.maximum(1.0 - jnp.exp(logp), 1e-20))
        g = jax.nn.sigmoid((logits + nm_ref[...]) / TEMPERATURE)
        ii = bi * BR + lax.broadcasted_iota(jnp.int32, (BR, BC), 0)
        jj = bj * BC + lax.broadcasted_iota(jnp.int32, (BR, BC), 1)
        o_ref[...] = jnp.where(ii < jj, g, 0.0)


_dense = pl.pallas_call(
    _dense_body,
    grid=(N // BR, N // BC),
    in_specs=[
        pl.BlockSpec(memory_space=pltpu.SMEM),
        pl.BlockSpec((BR, DIM_U), lambda i, j: (i, 0)),
        pl.BlockSpec((BC, DIM_U), lambda i, j: (j, 0)),
        pl.BlockSpec((BR, BC), lambda i, j: (i, j)),
    ],
    out_specs=pl.BlockSpec((BR, BC), lambda i, j: (i, j)),
    out_shape=jax.ShapeDtypeStruct((N, N), jnp.float32),
)


CH = 8


@functools.partial(
    pl.kernel,
    mesh=_MESH,
    compiler_params=_SC_PARAMS,
    out_type=jax.ShapeDtypeStruct((N * N,), jnp.float32),
    scratch_types=[
        pltpu.VMEM((N,), jnp.int32),
        pltpu.VMEM((2 * CH,), jnp.int32),
        pltpu.VMEM((2 * CH, N), jnp.float32),
        pltpu.VMEM((2 * CH * N,), jnp.float32),
        pltpu.SemaphoreType.DMA,
        pltpu.SemaphoreType.DMA,
    ],
)
def _unsort(s_hbm, rank_hbm, out_hbm, rank_v, idx_v, rows_v, orows_v,
            sem_in, sem_out):
    wid = lax.axis_index("s") * NC + lax.axis_index("c")
    base = pl.multiple_of(wid * ROWS_W, ROWS_W)
    NCHUNK = ROWS_W // CH

    pltpu.sync_copy(rank_hbm, rank_v)

    def islice(b):
        return idx_v.at[pl.ds(pl.multiple_of(b * CH, 8), CH)]

    def rrows(b):
        return rows_v.at[pl.ds(b * CH, CH)]

    def oslice(b):
        return orows_v.at[pl.ds(pl.multiple_of(b * CH * N, 8), CH * N)]

    def oout(c):
        return out_hbm.at[pl.ds(pl.multiple_of((base + c * CH) * N, 8),
                                CH * N)]

    def fetch(c, slot):
        pltpu.sync_copy(rank_hbm.at[pl.ds(base + c * CH, CH)], islice(slot))
        pltpu.async_copy(s_hbm.at[islice(slot)], rrows(slot), sem_in)

    fetch(0, 0)

    def chunk_body(c, carry):
        slot = lax.rem(c, 2)
        nslot = 1 - slot

        @pl.when(c + 1 < NCHUNK)
        def _():
            fetch(c + 1, nslot)

        pltpu.make_async_copy(
            s_hbm.at[islice(slot)], rrows(slot), sem_in).wait()

        @pl.when(c >= 2)
        def _():
            pltpu.make_async_copy(oslice(slot), oout(c - 2), sem_out).wait()

        dst = oslice(slot)
        rbase = slot * CH
        for k in range(CH):
            koff = k * N

            @plsc.parallel_loop(0, N, L, unroll=8)
            def _gather(o, _k=k, _koff=koff):
                cols = rank_v[pl.ds(o, L)]
                rowid = jnp.full((L,), rbase + _k, jnp.int32)
                dst[pl.ds(_koff + o, L)] = plsc.load_gather(
                    rows_v, [rowid, cols])

        pltpu.async_copy(dst, oout(c), sem_out)
        return carry

    lax.fori_loop(0, NCHUNK, chunk_body, 0)
    for k in range(2):
        pltpu.make_async_copy(oslice(k), oout(k), sem_out).wait()


def kernel(uR, g_logscale, noise):
    ordering = jnp.sum(jnp.log(0.5 + 0.5 * erf(uR / np.sqrt(2.0))),
                       axis=1, keepdims=True)
    sort_idx = jnp.argsort(jnp.squeeze(ordering))
    rank = (jnp.zeros((N,), jnp.int32)
            .at[sort_idx].set(jnp.arange(N, dtype=jnp.int32)))

    shear = _noise_shear(jnp.reshape(noise, (P,)))
    Y = uR[sort_idx, :]
    inv2s = (-0.5 * jnp.exp(-g_logscale)).reshape(1, 1)
    s_mat = _dense(inv2s, Y, Y, shear)
    return jnp.reshape(_unsort(s_mat, rank), (N, N))


# speedup vs baseline: 1.0282x; 1.0282x over previous
import functools

import jax
import jax.numpy as jnp
import numpy as np
from jax import lax
from jax.scipy.special import erf
from jax.experimental import pallas as pl
from jax.experimental.pallas import tpu as pltpu
from jax.experimental.pallas import tpu_sc as plsc

N = 2048
DIM_U = 16
TEMPERATURE = 0.3
LOG2 = 0.69314718056
P = N * (N - 1) // 2

NC, NS, L = 2, 16, 16
NW = NC * NS
ROWS_W = N // NW
WIN = N + 8
PPAD = ((P - N) // 8) * 8 + WIN

_MESH = plsc.VectorSubcoreMesh(core_axis_name="c", subcore_axis_name="s")


SPAN = 16384
GR = 8


def _vidx(i):
    return i * (N - 1) - ((i * (i - 1)) >> 1) - i - 1


@functools.partial(
    pl.kernel,
    mesh=_MESH,
    compiler_params=pltpu.CompilerParams(needs_layout_passes=False),
    out_type=jax.ShapeDtypeStruct((N * N,), jnp.float32),
    scratch_types=[
        pltpu.VMEM((N,), jnp.int32),
        pltpu.VMEM((2 * SPAN,), jnp.float32),
        pltpu.VMEM((2 * GR * N,), jnp.float32),
        pltpu.SemaphoreType.DMA,
        pltpu.SemaphoreType.DMA,
    ],
)
def _noise_gather(noise_hbm, rank_hbm, nm_hbm, rank_v, span_v, row_v,
                  sem_in, sem_out):
    wid = lax.axis_index("s") * NC + lax.axis_index("c")
    base = pl.multiple_of(wid * ROWS_W, ROWS_W)
    NG = ROWS_W // GR

    pltpu.sync_copy(rank_hbm, rank_v)

    def span_start(g):
        v0 = _vidx(base + g * GR)
        w8 = jnp.maximum(jnp.minimum(v0 & -8, P - SPAN), 0)
        return pl.multiple_of(w8, 8)

    def sslice(b):
        return span_v.at[pl.ds(pl.multiple_of(b * SPAN, 8), SPAN)]

    def gslice(b):
        return row_v.at[pl.ds(pl.multiple_of(b * GR * N, 8), GR * N)]

    def nm_out(g):
        return nm_hbm.at[pl.ds(pl.multiple_of((base + g * GR) * N, 8),
                               GR * N)]

    pltpu.async_copy(noise_hbm.at[pl.ds(span_start(0), SPAN)], sslice(0),
                     sem_in)

    def grp_body(g, carry):
        slot = lax.rem(g, 2)
        nslot = 1 - slot

        @pl.when(g + 1 < NG)
        def _():
            pltpu.async_copy(noise_hbm.at[pl.ds(span_start(g + 1), SPAN)],
                             sslice(nslot), sem_in)

        pltpu.make_async_copy(
            noise_hbm.at[pl.ds(0, SPAN)], sslice(slot), sem_in).wait()

        @pl.when(g >= 2)
        def _():
            pltpu.make_async_copy(gslice(slot), nm_out(g - 2),
                                  sem_out).wait()

        w8g = span_start(g)
        span = sslice(slot)
        grp = gslice(slot)
        for k in range(GR):
            dlt = _vidx(base + g * GR + k) - w8g

            @plsc.parallel_loop(0, N, L, unroll=8)
            def _gather(o, _dlt=dlt, _k=k):
                off = jnp.maximum(rank_v[pl.ds(o, L)] + _dlt, 0)
                grp[pl.ds(_k * N + o, L)] = plsc.load_gather(span, [off])

        pltpu.async_copy(grp, nm_out(g), sem_out)
        return carry

    lax.fori_loop(0, NG, grp_body, 0)
    for k in range(2):
        pltpu.make_async_copy(gslice(k), nm_out(k), sem_out).wait()


BR, BC = 256, 512


def _dense_body(s_ref, y_ref, u_ref, nm_ref, rk_ref, o_ref):
    inv2s = s_ref[0, 0]
    y = y_ref[...]
    u = u_ref[...]
    ny = jnp.sum(y * y, axis=1, keepdims=True)
    nu = jnp.sum(u * u, axis=1)[None, :]
    dot = lax.dot_general(y, u, (((1,), (1,)), ((), ())),
                          preferred_element_type=jnp.float32)
    d2 = jnp.maximum(ny + nu - 2.0 * dot, 0.0)
    logp = d2 * inv2s
    logits = logp - jnp.log(jnp.maximum(1.0 - jnp.exp(logp), 1e-20))
    g = jax.nn.sigmoid((logits + nm_ref[...]) / TEMPERATURE)
    ii = pl.program_id(0) * BR + lax.broadcasted_iota(jnp.int32, (BR, BC), 0)
    o_ref[...] = jnp.where(ii < rk_ref[0:1, :], g, 0.0)


_dense = pl.pallas_call(
    _dense_body,
    grid=(N // BR, N // BC),
    in_specs=[
        pl.BlockSpec(memory_space=pltpu.SMEM),
        pl.BlockSpec((BR, DIM_U), lambda i, j: (i, 0)),
        pl.BlockSpec((BC, DIM_U), lambda i, j: (j, 0)),
        pl.BlockSpec((BR, BC), lambda i, j: (i, j)),
        pl.BlockSpec((8, BC), lambda i, j: (0, j)),
    ],
    out_specs=pl.BlockSpec((BR, BC), lambda i, j: (i, j)),
    out_shape=jax.ShapeDtypeStruct((N, N), jnp.float32),
)


CH = 16


@functools.partial(
    pl.kernel,
    mesh=_MESH,
    out_type=jax.ShapeDtypeStruct((N, N), jnp.float32),
    scratch_types=[
        pltpu.VMEM((CH,), jnp.int32),
        pltpu.VMEM((CH, N), jnp.float32),
        pltpu.SemaphoreType.DMA,
    ],
)
def _row_permute(t1_hbm, rank_hbm, out_hbm, idx_v, rows_v, sem):
    wid = lax.axis_index("s") * NC + lax.axis_index("c")
    base = pl.multiple_of(wid * ROWS_W, ROWS_W)
    for c in range(ROWS_W // CH):
        pltpu.sync_copy(rank_hbm.at[pl.ds(base + c * CH, CH)], idx_v)
        pltpu.async_copy(t1_hbm.at[idx_v], rows_v, sem).wait()
        pltpu.sync_copy(rows_v, out_hbm.at[pl.ds(base + c * CH, CH)])


def kernel(uR, g_logscale, noise):
    ordering = jnp.sum(jnp.log(0.5 + 0.5 * erf(uR / np.sqrt(2.0))),
                       axis=1, keepdims=True)
    sort_idx = jnp.argsort(jnp.squeeze(ordering))
    rank = (jnp.zeros((N,), jnp.int32)
            .at[sort_idx].set(jnp.arange(N, dtype=jnp.int32)))

    nm1 = jnp.reshape(_noise_gather(jnp.reshape(noise, (P,)), rank), (N, N))
    Y = uR[sort_idx, :]
    inv2s = (-0.5 * jnp.exp(-g_logscale)).reshape(1, 1)
    rk8 = jnp.broadcast_to(rank[None, :], (8, N))
    t1 = _dense(inv2s, Y, uR, nm1, rk8)
    return _row_permute(t1, rank)


# speedup vs baseline: 1.1388x; 1.1076x over previous
import functools

import jax
import jax.numpy as jnp
import numpy as np
from jax import lax
from jax.scipy.special import erf
from jax.experimental import pallas as pl
from jax.experimental.pallas import tpu as pltpu
from jax.experimental.pallas import tpu_sc as plsc

N = 2048
DIM_U = 16
TEMPERATURE = 0.3
LOG2 = 0.69314718056
P = N * (N - 1) // 2

NC, NS, L = 2, 16, 16
NW = NC * NS
ROWS_W = N // NW
WIN = N + 8
PPAD = ((P - N) // 8) * 8 + WIN

_MESH = plsc.VectorSubcoreMesh(core_axis_name="c", subcore_axis_name="s")


@functools.partial(
    pl.kernel,
    mesh=_MESH,
    compiler_params=pltpu.CompilerParams(needs_layout_passes=False),
    out_type=jax.ShapeDtypeStruct((N, N), jnp.float32),
    scratch_types=[
        pltpu.VMEM((N,), jnp.int32),
        pltpu.VMEM((4 * WIN,), jnp.float32),
        pltpu.VMEM((4 * N,), jnp.float32),
        pltpu.SemaphoreType.DMA,
        pltpu.SemaphoreType.DMA,
    ],
)
def _noise_gather(noise_hbm, rank_hbm, nm_hbm, rank_v, win_v, row_v,
                  sem_in, sem_out):
    wid = lax.axis_index("s") * NC + lax.axis_index("c")
    base = pl.multiple_of(wid * ROWS_W, ROWS_W)

    pltpu.sync_copy(rank_hbm, rank_v)

    def win_start(i):
        v = i * (N - 1) - ((i * (i - 1)) >> 1) - i - 1
        w8 = jnp.maximum(jnp.minimum(v & -8, P - WIN), 0)
        w8 = pl.multiple_of(w8, 8)
        return w8, v - w8

    RING = 4

    def wslice(b):
        return win_v.at[pl.ds(pl.multiple_of(b * WIN, 8), WIN)]

    def rslice(b):
        return row_v.at[pl.ds(pl.multiple_of(b * N, 8), N)]

    for k in range(RING - 1):
        w8k, _ = win_start(base + k)
        pltpu.async_copy(noise_hbm.at[pl.ds(w8k, WIN)], wslice(k), sem_in)

    def row_body(r, carry):
        i = base + r
        slot = lax.rem(r, RING)
        pslot = lax.rem(r + RING - 1, RING)
        w8n, _ = win_start(i + RING - 1)
        pltpu.async_copy(noise_hbm.at[pl.ds(w8n, WIN)], wslice(pslot), sem_in)
        pltpu.make_async_copy(
            noise_hbm.at[pl.ds(w8n, WIN)], wslice(slot), sem_in).wait()

        @pl.when(r >= RING)
        def _():
            pltpu.make_async_copy(rslice(slot), nm_hbm.at[i - RING],
                                  sem_out).wait()

        _, dlt = win_start(i)
        win = wslice(slot)
        row = rslice(slot)

        @plsc.parallel_loop(0, N, L, unroll=8)
        def _gather(o):
            off = jnp.maximum(rank_v[pl.ds(o, L)] + dlt, 0)
            row[pl.ds(o, L)] = plsc.load_gather(win, [off])
        pltpu.async_copy(row, nm_hbm.at[i], sem_out)
        return carry

    lax.fori_loop(0, ROWS_W, row_body, 0)
    for k in range(RING - 1):
        pltpu.make_async_copy(
            noise_hbm.at[pl.ds(0, WIN)], wslice(k), sem_in).wait()
    for k in range(RING):
        pltpu.make_async_copy(rslice(k), nm_hbm.at[base + k], sem_out).wait()


BR, BC = 256, 512


def _dense_body(s_ref, y_ref, u_ref, nm_ref, rk_ref, o_ref):
    inv2s = s_ref[0, 0]
    y = y_ref[...]
    u = u_ref[...]
    ny = jnp.sum(y * y, axis=1, keepdims=True)
    nu = jnp.sum(u * u, axis=1)[None, :]
    dot = lax.dot_general(y, u, (((1,), (1,)), ((), ())),
                          preferred_element_type=jnp.float32)
    d2 = jnp.maximum(ny + nu - 2.0 * dot, 0.0)
    logp = d2 * inv2s
    logits = logp - jnp.log(jnp.maximum(1.0 - jnp.exp(logp), 1e-20))
    g = jax.nn.sigmoid((logits + nm_ref[...]) / TEMPERATURE)
    ii = pl.program_id(0) * BR + lax.broadcasted_iota(jnp.int32, (BR, BC), 0)
    o_ref[...] = jnp.where(ii < rk_ref[0:1, :], g, 0.0)


_dense = pl.pallas_call(
    _dense_body,
    grid=(N // BR, N // BC),
    in_specs=[
        pl.BlockSpec(memory_space=pltpu.SMEM),
        pl.BlockSpec((BR, DIM_U), lambda i, j: (i, 0)),
        pl.BlockSpec((BC, DIM_U), lambda i, j: (j, 0)),
        pl.BlockSpec((BR, BC), lambda i, j: (i, j)),
        pl.BlockSpec((8, BC), lambda i, j: (0, j)),
    ],
    out_specs=pl.BlockSpec((BR, BC), lambda i, j: (i, j)),
    out_shape=jax.ShapeDtypeStruct((N, N), jnp.float32),
)


CH = 16


@functools.partial(
    pl.kernel,
    mesh=_MESH,
    out_type=jax.ShapeDtypeStruct((N, N), jnp.float32),
    scratch_types=[
        pltpu.VMEM((CH,), jnp.int32),
        pltpu.VMEM((CH, N), jnp.float32),
        pltpu.SemaphoreType.DMA,
    ],
)
def _row_permute(t1_hbm, rank_hbm, out_hbm, idx_v, rows_v, sem):
    wid = lax.axis_index("s") * NC + lax.axis_index("c")
    base = pl.multiple_of(wid * ROWS_W, ROWS_W)
    for c in range(ROWS_W // CH):
        pltpu.sync_copy(rank_hbm.at[pl.ds(base + c * CH, CH)], idx_v)
        pltpu.async_copy(t1_hbm.at[idx_v], rows_v, sem).wait()
        pltpu.sync_copy(rows_v, out_hbm.at[pl.ds(base + c * CH, CH)])


def kernel(uR, g_logscale, noise):
    ordering = jnp.sum(jnp.log(0.5 + 0.5 * erf(uR / np.sqrt(2.0))),
                       axis=1, keepdims=True)
    sort_idx = jnp.argsort(jnp.squeeze(ordering))
    rank = (jnp.zeros((N,), jnp.int32)
            .at[sort_idx].set(jnp.arange(N, dtype=jnp.int32)))

    nm1 = _noise_gather(jnp.reshape(noise, (P,)), rank)
    Y = uR[sort_idx, :]
    inv2s = (-0.5 * jnp.exp(-g_logscale)).reshape(1, 1)
    rk8 = jnp.broadcast_to(rank[None, :], (8, N))
    t1 = _dense(inv2s, Y, uR, nm1, rk8)
    return _row_permute(t1, rank)
